# baseline (speedup 1.0000x reference)
import functools

import jax
import jax.numpy as jnp
from jax import lax
from jax.experimental import pallas as pl
from jax.experimental.pallas import tpu as pltpu
from jax.experimental.pallas import tpu_sc as plsc

R, C = 128, 32768
NC, NS, L = 2, 16, 16
NW = NC * NS
RPB = 8
W_SC = 16384
COLS = W_SC // 2
CW = 2048
NCH = COLS // CW
J1 = CW // L
J2 = COLS // L

TCB = 1024
TC_C0 = W_SC // TCB
TC_G = (C - W_SC) // TCB

_I32_MAX = jnp.iinfo(jnp.int32).max


@functools.partial(
    pl.kernel,
    out_type=(
        jax.ShapeDtypeStruct((NW * L,), jnp.float32),
        jax.ShapeDtypeStruct((NW * L,), jnp.int32),
    ),
    mesh=plsc.VectorSubcoreMesh(core_axis_name="c", subcore_axis_name="s"),
    scratch_types=(
        pltpu.VMEM((RPB, COLS), jnp.float32),
        pltpu.VMEM((L,), jnp.float32),
        pltpu.VMEM((L,), jnp.int32),
        [pltpu.SemaphoreType.DMA] * NCH,
    ),
    compiler_params=pltpu.CompilerParams(
        use_tc_tiling_on_sc=True, needs_layout_passes=False,
        skip_device_barrier=True),
)
def _stage1_sc(x_hbm, outv_hbm, outi_hbm, buf, ov, oi, sems):
    c = lax.axis_index("c")
    s = lax.axis_index("s")
    wid = s * NC + c
    row0 = (wid // 2) * RPB
    c0 = (wid % 2) * COLS

    copies = []
    for ci in range(NCH):
        copies.append(pltpu.async_copy(
            x_hbm.at[pl.ds(row0, RPB), pl.ds(c0 + ci * CW, CW)],
            buf.at[:, pl.ds(ci * CW, CW)], sems[ci]))

    neg_inf = jnp.full((L,), -jnp.inf, jnp.float32)
    vmaxs = (neg_inf,) * RPB
    for ci in range(NCH):
        copies[ci].wait()

        def body(j, vm):
            col = ci * CW + j * L
            return tuple(
                jnp.maximum(vm[r], buf[r, pl.ds(col, L)]) for r in range(RPB))

        vmaxs = lax.fori_loop(0, J1, body, vmaxs, unroll=2)

    rms = [jnp.max(vmaxs[r]) for r in range(RPB)]
    m = rms[0]
    for r in range(1, RPB):
        m = jnp.maximum(m, rms[r])
    rstar = jnp.int32(RPB - 1)
    for r in range(RPB - 2, -1, -1):
        rstar = jnp.where(rms[r] == m, jnp.int32(r), rstar)

    mvec = jnp.full((L,), m, jnp.float32)
    big = jnp.full((L,), _I32_MAX, jnp.int32)
    vstep = big
    for r in range(RPB):
        @pl.when(rstar == r)
        def _():
            def body(j, vs):
                x = buf[r, pl.ds(j * L, L)]
                return jnp.minimum(
                    vs, jnp.where(x == mvec, jnp.full((L,), j, jnp.int32), big))

            found = lax.fori_loop(0, J2, body, vstep, unroll=2)
            lane = lax.broadcasted_iota(jnp.int32, (L,), 0)
            sel = jnp.where(found != big, found * L + lane, big)
            colmin = jnp.min(sel)
            flat = (row0 + rstar) * C + c0 + colmin
            ov[...] = mvec
            oi[...] = jnp.full((L,), flat, jnp.int32)

    pltpu.sync_copy(ov, outv_hbm.at[pl.ds(wid * L, L)])
    pltpu.sync_copy(oi, outi_hbm.at[pl.ds(wid * L, L)])


def _stage1_tc_body(x_ref, om_ref, oi_ref, idxc, sm, si):
    g = pl.program_id(0)
    x = x_ref[...]
    m = jnp.max(x)

    @pl.when(g == 0)
    def _():
        rows = lax.broadcasted_iota(jnp.int32, (R, TCB), 0)
        cols = lax.broadcasted_iota(jnp.int32, (R, TCB), 1)
        idxc[...] = rows * C + cols
        li = jnp.min(jnp.where(x == m, idxc[...], _I32_MAX)) + W_SC
        sm[0] = m
        si[0] = li

    @pl.when(g > 0)
    def _():
        @pl.when(m >= sm[0])
        def _():
            li = jnp.min(jnp.where(x == m, idxc[...], _I32_MAX)) + (
                W_SC + g * TCB)
            better = (m > sm[0]) | (li < si[0])

            @pl.when(better)
            def _():
                sm[0] = m
                si[0] = li

    @pl.when(g == TC_G - 1)
    def _():
        om_ref[0, 0] = sm[0]
        oi_ref[0, 0] = si[0]


_stage1_tc = pl.pallas_call(
    _stage1_tc_body,
    grid=(TC_G,),
    in_specs=[pl.BlockSpec((R, TCB), lambda g: (0, TC_C0 + g))],
    out_specs=(
        pl.BlockSpec(memory_space=pltpu.SMEM),
        pl.BlockSpec(memory_space=pltpu.SMEM),
    ),
    out_shape=(
        jax.ShapeDtypeStruct((1, 1), jnp.float32),
        jax.ShapeDtypeStruct((1, 1), jnp.int32),
    ),
    scratch_shapes=[
        pltpu.VMEM((R, TCB), jnp.int32),
        pltpu.SMEM((1,), jnp.float32),
        pltpu.SMEM((1,), jnp.int32),
    ],
)


def _merge_body(v_ref, i_ref, tm_ref, ti_ref, o_ref):
    v = v_ref[...]
    idx = i_ref[...]
    tm = tm_ref[0, 0]
    ti = ti_ref[0, 0]
    m = jnp.maximum(jnp.max(v), tm)
    best_sc = jnp.min(jnp.where(v == m, idx, _I32_MAX))
    o_ref[0, 0] = jnp.where(tm == m, jnp.minimum(best_sc, ti), best_sc)


_merge = pl.pallas_call(
    _merge_body,
    in_specs=[
        pl.BlockSpec(memory_space=pltpu.VMEM),
        pl.BlockSpec(memory_space=pltpu.VMEM),
        pl.BlockSpec(memory_space=pltpu.VMEM),
        pl.BlockSpec(memory_space=pltpu.VMEM),
    ],
    out_shape=jax.ShapeDtypeStruct((1, 1), jnp.int32),
    out_specs=pl.BlockSpec(memory_space=pltpu.SMEM),
)


@jax.jit
def kernel(input):
    pv, pi = _stage1_sc(input)
    tm, ti = _stage1_tc(input)
    return _merge(pv.reshape(4, 128), pi.reshape(4, 128), tm, ti)


# speedup vs baseline: 1.1299x; 1.1299x over previous
import functools

import jax
import jax.numpy as jnp
import numpy as np
from jax import lax
from jax.experimental import pallas as pl
from jax.experimental.pallas import tpu as pltpu
from jax.experimental.pallas import tpu_sc as plsc

R, C = 128, 32768
NC, NS, L = 2, 16, 16
NW = NC * NS
RPB = 8
W_SC = 12288
COLS = W_SC // 2
CW = 3072
NCH = COLS // CW
J1 = CW // L

TCB = 4096
TC_C0 = W_SC // TCB
TC_G = (C - W_SC) // TCB

_I32_MAX = jnp.iinfo(jnp.int32).max

_IDX_LOCAL = jnp.asarray(
    np.arange(R, dtype=np.int32)[:, None] * C
    + np.arange(TCB, dtype=np.int32)[None, :])


def _scan_chunk(buf, ci, carry):
    def body(j, cr):
        vcnt = cr[0]
        vmaxs = list(cr[1:1 + RPB])
        vsteps = list(cr[1 + RPB:1 + 2 * RPB])
        for r in range(RPB):
            x = buf[r, pl.ds(ci * CW + j * L, L)]
            m = x > vmaxs[r]
            vmaxs[r] = jnp.where(m, x, vmaxs[r])
            vsteps[r] = jnp.where(m, vcnt, vsteps[r])
        return (vcnt + 1, *vmaxs, *vsteps)

    return lax.fori_loop(0, J1, body, carry, unroll=2)


@functools.partial(
    pl.kernel,
    out_type=(
        jax.ShapeDtypeStruct((NW * L,), jnp.float32),
        jax.ShapeDtypeStruct((NW * L,), jnp.int32),
    ),
    mesh=plsc.VectorSubcoreMesh(core_axis_name="c", subcore_axis_name="s"),
    scratch_types=(
        pltpu.VMEM((RPB, COLS), jnp.float32),
        pltpu.VMEM((L,), jnp.float32),
        pltpu.VMEM((L,), jnp.int32),
        [pltpu.SemaphoreType.DMA] * NCH,
    ),
    compiler_params=pltpu.CompilerParams(
        use_tc_tiling_on_sc=True, needs_layout_passes=False),
)
def _stage1_sc(x_hbm, outv_hbm, outi_hbm, buf, ov, oi, sems):
    c = lax.axis_index("c")
    s = lax.axis_index("s")
    wid = s * NC + c
    row0 = (wid // 2) * RPB
    c0 = (wid % 2) * COLS

    copies = []
    for ci in range(NCH):
        copies.append(pltpu.async_copy(
            x_hbm.at[pl.ds(row0, RPB), pl.ds(c0 + ci * CW, CW)],
            buf.at[:, pl.ds(ci * CW, CW)], sems[ci]))

    neg_inf = jnp.full((L,), -jnp.inf, jnp.float32)
    zero = jnp.zeros((L,), jnp.int32)
    carry = (zero,) + (neg_inf,) * RPB + (zero,) * RPB
    for ci in range(NCH):
        copies[ci].wait()
        carry = _scan_chunk(buf, ci, carry)

    vmaxs = carry[1:1 + RPB]
    vsteps = carry[1 + RPB:1 + 2 * RPB]
    lane = lax.broadcasted_iota(jnp.int32, (L,), 0)

    vm = vmaxs[0]
    vi = (row0 + 0) * C + c0 + vsteps[0] * L + lane
    for r in range(1, RPB):
        mb = vmaxs[r]
        ib = (row0 + r) * C + c0 + vsteps[r] * L + lane
        take = (mb > vm) | ((mb == vm) & (ib < vi))
        vm = jnp.where(take, mb, vm)
        vi = jnp.where(take, ib, vi)

    ov[...] = vm
    oi[...] = vi
    pltpu.sync_copy(ov, outv_hbm.at[pl.ds(wid * L, L)])
    pltpu.sync_copy(oi, outi_hbm.at[pl.ds(wid * L, L)])


def _stage1_tc_body(x_ref, idx_ref, om_ref, oi_ref, sm, si):
    g = pl.program_id(0)
    x = x_ref[...]
    m = jnp.max(x)

    @pl.when(g == 0)
    def _():
        li = jnp.min(jnp.where(x == m, idx_ref[...], _I32_MAX)) + W_SC
        sm[0] = m
        si[0] = li

    @pl.when(g > 0)
    def _():
        @pl.when(m >= sm[0])
        def _():
            li = jnp.min(jnp.where(x == m, idx_ref[...], _I32_MAX)) + (
                W_SC + g * TCB)
            better = (m > sm[0]) | (li < si[0])

            @pl.when(better)
            def _():
                sm[0] = m
                si[0] = li

    @pl.when(g == TC_G - 1)
    def _():
        om_ref[0, 0] = sm[0]
        oi_ref[0, 0] = si[0]


_stage1_tc = pl.pallas_call(
    _stage1_tc_body,
    grid=(TC_G,),
    in_specs=[
        pl.BlockSpec((R, TCB), lambda g: (0, TC_C0 + g)),
        pl.BlockSpec((R, TCB), lambda g: (0, 0)),
    ],
    out_specs=(
        pl.BlockSpec(memory_space=pltpu.SMEM),
        pl.BlockSpec(memory_space=pltpu.SMEM),
    ),
    out_shape=(
        jax.ShapeDtypeStruct((1, 1), jnp.float32),
        jax.ShapeDtypeStruct((1, 1), jnp.int32),
    ),
    scratch_shapes=[
        pltpu.SMEM((1,), jnp.float32),
        pltpu.SMEM((1,), jnp.int32),
    ],
)


def _merge_body(v_ref, i_ref, tm_ref, ti_ref, o_ref):
    v = v_ref[...]
    idx = i_ref[...]
    tm = tm_ref[0, 0]
    ti = ti_ref[0, 0]
    m = jnp.maximum(jnp.max(v), tm)
    best_sc = jnp.min(jnp.where(v == m, idx, _I32_MAX))
    o_ref[0, 0] = jnp.where(tm == m, jnp.minimum(best_sc, ti), best_sc)


_merge = pl.pallas_call(
    _merge_body,
    in_specs=[
        pl.BlockSpec(memory_space=pltpu.VMEM),
        pl.BlockSpec(memory_space=pltpu.VMEM),
        pl.BlockSpec(memory_space=pltpu.SMEM),
        pl.BlockSpec(memory_space=pltpu.SMEM),
    ],
    out_shape=jax.ShapeDtypeStruct((1, 1), jnp.int32),
    out_specs=pl.BlockSpec(memory_space=pltpu.SMEM),
)


@jax.jit
def kernel(input):
    pv, pi = _stage1_sc(input)
    tm, ti = _stage1_tc(input, _IDX_LOCAL)
    return _merge(pv.reshape(4, 128), pi.reshape(4, 128), tm, ti)


# speedup vs baseline: 1.1382x; 1.0074x over previous
import functools

import jax
import jax.numpy as jnp
import numpy as np
from jax import lax
from jax.experimental import pallas as pl
from jax.experimental.pallas import tpu as pltpu
from jax.experimental.pallas import tpu_sc as plsc

R, C = 128, 32768
NC, NS, L = 2, 16, 16
NW = NC * NS
RPB = 8
W_SC = 12288
COLS = W_SC // 2
CW = 3072
NCH = COLS // CW
J1 = CW // L

TCB = 4096
TC_C0 = W_SC // TCB
TC_G = (C - W_SC) // TCB

_I32_MAX = jnp.iinfo(jnp.int32).max


def _scan_chunk(buf, ci, carry):
    def body(j, cr):
        vcnt = cr[0]
        vmaxs = list(cr[1:1 + RPB])
        vsteps = list(cr[1 + RPB:1 + 2 * RPB])
        for r in range(RPB):
            x = buf[r, pl.ds(ci * CW + j * L, L)]
            m = x > vmaxs[r]
            vmaxs[r] = jnp.where(m, x, vmaxs[r])
            vsteps[r] = jnp.where(m, vcnt, vsteps[r])
        return (vcnt + 1, *vmaxs, *vsteps)

    return lax.fori_loop(0, J1, body, carry, unroll=2)


@functools.partial(
    pl.kernel,
    out_type=(
        jax.ShapeDtypeStruct((NW * L,), jnp.float32),
        jax.ShapeDtypeStruct((NW * L,), jnp.int32),
    ),
    mesh=plsc.VectorSubcoreMesh(core_axis_name="c", subcore_axis_name="s"),
    scratch_types=(
        pltpu.VMEM((RPB, COLS), jnp.float32),
        pltpu.VMEM((L,), jnp.float32),
        pltpu.VMEM((L,), jnp.int32),
        [pltpu.SemaphoreType.DMA] * NCH,
    ),
    compiler_params=pltpu.CompilerParams(
        use_tc_tiling_on_sc=True, needs_layout_passes=False),
)
def _stage1_sc(x_hbm, outv_hbm, outi_hbm, buf, ov, oi, sems):
    c = lax.axis_index("c")
    s = lax.axis_index("s")
    wid = s * NC + c
    row0 = (wid // 2) * RPB
    c0 = (wid % 2) * COLS

    copies = []
    for ci in range(NCH):
        copies.append(pltpu.async_copy(
            x_hbm.at[pl.ds(row0, RPB), pl.ds(c0 + ci * CW, CW)],
            buf.at[:, pl.ds(ci * CW, CW)], sems[ci]))

    neg_inf = jnp.full((L,), -jnp.inf, jnp.float32)
    zero = jnp.zeros((L,), jnp.int32)
    carry = (zero,) + (neg_inf,) * RPB + (zero,) * RPB
    for ci in range(NCH):
        copies[ci].wait()
        carry = _scan_chunk(buf, ci, carry)

    vmaxs = carry[1:1 + RPB]
    vsteps = carry[1 + RPB:1 + 2 * RPB]
    lane = lax.broadcasted_iota(jnp.int32, (L,), 0)

    vm = vmaxs[0]
    vi = (row0 + 0) * C + c0 + vsteps[0] * L + lane
    for r in range(1, RPB):
        mb = vmaxs[r]
        ib = (row0 + r) * C + c0 + vsteps[r] * L + lane
        take = (mb > vm) | ((mb == vm) & (ib < vi))
        vm = jnp.where(take, mb, vm)
        vi = jnp.where(take, ib, vi)

    ov[...] = vm
    oi[...] = vi
    pltpu.sync_copy(ov, outv_hbm.at[pl.ds(wid * L, L)])
    pltpu.sync_copy(oi, outi_hbm.at[pl.ds(wid * L, L)])


def _stripe_min_idx(x, m):
    rows = lax.broadcasted_iota(jnp.int32, (R, TCB), 0)
    cols = lax.broadcasted_iota(jnp.int32, (R, TCB), 1)
    return jnp.min(jnp.where(x == m, rows * C + cols, _I32_MAX))


def _stage1_tc_body(x_ref, om_ref, oi_ref, sm, si):
    g = pl.program_id(0)
    x = x_ref[...]
    m = jnp.max(x)

    @pl.when(g == 0)
    def _():
        sm[0] = m
        si[0] = _stripe_min_idx(x, m) + W_SC

    @pl.when(g > 0)
    def _():
        @pl.when(m >= sm[0])
        def _():
            li = _stripe_min_idx(x, m) + (W_SC + g * TCB)
            better = (m > sm[0]) | (li < si[0])

            @pl.when(better)
            def _():
                sm[0] = m
                si[0] = li

    @pl.when(g == TC_G - 1)
    def _():
        om_ref[0, 0] = sm[0]
        oi_ref[0, 0] = si[0]


_stage1_tc = pl.pallas_call(
    _stage1_tc_body,
    grid=(TC_G,),
    in_specs=[
        pl.BlockSpec((R, TCB), lambda g: (0, TC_C0 + g)),
    ],
    out_specs=(
        pl.BlockSpec(memory_space=pltpu.SMEM),
        pl.BlockSpec(memory_space=pltpu.SMEM),
    ),
    out_shape=(
        jax.ShapeDtypeStruct((1, 1), jnp.float32),
        jax.ShapeDtypeStruct((1, 1), jnp.int32),
    ),
    scratch_shapes=[
        pltpu.SMEM((1,), jnp.float32),
        pltpu.SMEM((1,), jnp.int32),
    ],
)


def _merge_body(v_ref, i_ref, tm_ref, ti_ref, o_ref):
    v = v_ref[...]
    idx = i_ref[...]
    tm = tm_ref[0, 0]
    ti = ti_ref[0, 0]
    m = jnp.maximum(jnp.max(v), tm)
    best_sc = jnp.min(jnp.where(v == m, idx, _I32_MAX))
    o_ref[0, 0] = jnp.where(tm == m, jnp.minimum(best_sc, ti), best_sc)


_merge = pl.pallas_call(
    _merge_body,
    in_specs=[
        pl.BlockSpec(memory_space=pltpu.VMEM),
        pl.BlockSpec(memory_space=pltpu.VMEM),
        pl.BlockSpec(memory_space=pltpu.SMEM),
        pl.BlockSpec(memory_space=pltpu.SMEM),
    ],
    out_shape=jax.ShapeDtypeStruct((1, 1), jnp.int32),
    out_specs=pl.BlockSpec(memory_space=pltpu.SMEM),
)


@jax.jit
def kernel(input):
    pv, pi = _stage1_sc(input)
    tm, ti = _stage1_tc(input)
    return _merge(pv.reshape(4, 128), pi.reshape(4, 128), tm, ti)
